# baseline (speedup 1.0000x reference)
import functools

import jax
import jax.numpy as jnp
from jax import lax
from jax.experimental import pallas as pl
from jax.experimental.pallas import tpu as pltpu
from jax.experimental.pallas import tpu_sc as plsc

B = 16
N = 4096
M = 128
KNN = 32
ROWS = B * M * KNN


def _fps_body(x_ref, y_ref, z_ref, cx_ref, cy_ref, cz_ref):
    x = x_ref[...]
    y = y_ref[...]
    z = z_ref[...]
    pt = jax.lax.broadcasted_iota(jnp.int32, (B, N), 1)

    def body(i, carry):
        dist, far = carry
        cm = pt == far[:, None]
        cx = jnp.sum(jnp.where(cm, x, 0.0), axis=1)
        cy = jnp.sum(jnp.where(cm, y, 0.0), axis=1)
        cz = jnp.sum(jnp.where(cm, z, 0.0), axis=1)
        cx_ref[pl.ds(i, 1), :] = cx[None, :]
        cy_ref[pl.ds(i, 1), :] = cy[None, :]
        cz_ref[pl.ds(i, 1), :] = cz[None, :]
        dx = x - cx[:, None]
        dy = y - cy[:, None]
        dz = z - cz[:, None]
        d = dx * dx + dy * dy + dz * dz
        dist = jnp.minimum(dist, d)
        rm = jnp.max(dist, axis=1)
        far = jnp.min(jnp.where(dist == rm[:, None], pt, N), axis=1).astype(jnp.int32)
        return dist, far

    dist0 = jnp.full((B, N), jnp.inf, dtype=jnp.float32)
    far0 = jnp.zeros((B,), dtype=jnp.int32)
    jax.lax.fori_loop(0, M, body, (dist0, far0))


def _run_fps(xt):
    out = pl.pallas_call(
        _fps_body,
        out_shape=[jax.ShapeDtypeStruct((M, B), jnp.float32)] * 3,
    )(xt[0], xt[1], xt[2])
    return out


def _dist_body(x_ref, y_ref, z_ref, cx_ref, cy_ref, cz_ref, d_ref):
    x = x_ref[0]
    y = y_ref[0]
    z = z_ref[0]
    cx = cx_ref[0]
    cy = cy_ref[0]
    cz = cz_ref[0]
    dxx = cx - x
    dyy = cy - y
    dzz = cz - z
    d_ref[0] = dxx * dxx + dyy * dyy + dzz * dzz


def _run_dist(xt, cxt):
    grid = (B,)
    pspec = pl.BlockSpec((1, 1, N), lambda b: (b, 0, 0))
    cspec = pl.BlockSpec((1, M, 1), lambda b: (b, 0, 0))
    return pl.pallas_call(
        _dist_body,
        grid=grid,
        in_specs=[pspec] * 3 + [cspec] * 3,
        out_specs=pl.BlockSpec((1, M, N), lambda b: (b, 0, 0)),
        out_shape=jax.ShapeDtypeStruct((B, M, N), jnp.float32),
    )(xt[0], xt[1], xt[2], cxt[0], cxt[1], cxt[2])


_ROWS_W = 64
_GRP = 8
_NBKT = 4096


def _knn_sc_body(d_hbm, x_hbm, y_hbm, z_hbm, cx_hbm, cy_hbm, cz_hbm,
                 px_hbm, py_hbm, pz_hbm,
                 dbuf, xbuf, ybuf, zbuf, cxb, cyb, czb,
                 hist, idxb, eqv, eqi, pxs, pys, pzs):
    wid = lax.axis_index("s") * 2 + lax.axis_index("c")
    b = wid // 2
    m0 = (wid % 2) * _ROWS_W
    io = lax.iota(jnp.int32, 16)
    ones16 = jnp.ones((16,), jnp.int32)
    zero16 = jnp.zeros((16,), jnp.int32)
    inf16 = jnp.full((16,), jnp.inf, jnp.float32)

    pltpu.sync_copy(x_hbm.at[b], xbuf)
    pltpu.sync_copy(y_hbm.at[b], ybuf)
    pltpu.sync_copy(z_hbm.at[b], zbuf)
    pltpu.sync_copy(cx_hbm.at[b, pl.ds(m0, _ROWS_W)], cxb)
    pltpu.sync_copy(cy_hbm.at[b, pl.ds(m0, _ROWS_W)], cyb)
    pltpu.sync_copy(cz_hbm.at[b, pl.ds(m0, _ROWS_W)], czb)

    def group(g, _g):
        pltpu.sync_copy(d_hbm.at[b, pl.ds(m0 + g * _GRP, _GRP)], dbuf)

        def row(rr, _r):
            rl = g * _GRP + rr

            @plsc.parallel_loop(0, 128, unroll=8, carry=(inf16, inf16))
            def scan1(c, carry):
                a, bb = carry
                va = dbuf[rr, pl.ds(pl.multiple_of(c * 16, 16), 16)]
                vb = dbuf[rr, pl.ds(pl.multiple_of((c + 128) * 16, 16), 16)]
                return jnp.minimum(a, va), jnp.minimum(bb, vb)

            accA, accB = scan1
            minv = jnp.min(jnp.minimum(accA, accB))
            q = jnp.maximum(jnp.max(accA), jnp.max(accB))
            den = jnp.maximum(q - minv, jnp.float32(1e-30))
            s = jnp.full((16,), 4094.0, jnp.float32) / jnp.full(
                (16,), den, jnp.float32)

            @plsc.parallel_loop(0, _NBKT // 16, unroll=8)
            def _clr(c):
                hist[pl.ds(pl.multiple_of(c * 16, 16), 16)] = zero16

            @plsc.parallel_loop(0, 256, unroll=8)
            def _hp(c):
                v = dbuf[rr, pl.ds(pl.multiple_of(c * 16, 16), 16)]
                bi = jnp.minimum((v - minv) * s, 4095.0).astype(jnp.int32)
                plsc.addupdate_scatter(hist, [bi], ones16)

            def condf(carry):
                j, tot, found, beta, below = carry
                return jnp.logical_and(found == 0, j < _NBKT // 16)

            def bodyf(carry):
                j, tot, found, beta, below = carry
                hv = hist[pl.ds(pl.multiple_of(j * 16, 16), 16)]
                pc = plsc.cumsum(hv)
                ta = pc + tot
                cr = ta >= KNN
                anyc = jnp.any(cr)
                l = jnp.min(jnp.where(cr, io, 16))
                below_c = jnp.maximum(jnp.max(jnp.where(cr, 0, ta)), tot)
                beta = jnp.where(anyc, j * 16 + l, beta)
                below = jnp.where(anyc, below_c, below)
                found = jnp.where(anyc, 1, found)
                return j + 1, jnp.max(ta), found, beta, below

            _, _, _, beta, below = lax.while_loop(
                condf, bodyf, (0, 0, 0, 0, 0))

            eqv[...] = inf16
            eqi[...] = zero16

            @plsc.parallel_loop(0, 256, unroll=4,
                                carry=(jnp.int32(0), jnp.int32(0)))
            def _cp(c, carry):
                v = dbuf[rr, pl.ds(pl.multiple_of(c * 16, 16), 16)]
                bi = jnp.minimum((v - minv) * s, 4095.0).astype(jnp.int32)

                def heavy(args):
                    bc, ec = args
                    pidx = io + c * 16
                    mlow = bi < beta
                    pcl = plsc.cumsum(mlow.astype(jnp.int32))
                    plsc.store_scatter(idxb, [pcl - 1 + bc], pidx, mask=mlow)
                    meq = bi == beta
                    pce = plsc.cumsum(meq.astype(jnp.int32))
                    pose = pce - 1 + ec
                    meq2 = jnp.logical_and(meq, pose < 16)
                    plsc.store_scatter(eqv, [pose], v, mask=meq2)
                    plsc.store_scatter(eqi, [pose], pidx, mask=meq2)
                    return bc + jnp.max(pcl), ec + jnp.max(pce)

                return lax.cond(jnp.min(bi) <= beta, heavy,
                                lambda a: a, carry)

            sv, si = plsc.sort_key_val(eqv[...], eqi[...])
            plsc.store_scatter(idxb, [io + below], si,
                               mask=io < (KNN - below))

            rls = jnp.full((16,), rl, jnp.int32)
            cxs = plsc.load_gather(cxb, [rls])
            cys = plsc.load_gather(cyb, [rls])
            czs = plsc.load_gather(czb, [rls])
            i0 = idxb[pl.ds(0, 16)]
            i1 = idxb[pl.ds(16, 16)]
            plsc.store_scatter(pxs, [rls, io], plsc.load_gather(xbuf, [i0]) - cxs)
            plsc.store_scatter(pxs, [rls, io + 16], plsc.load_gather(xbuf, [i1]) - cxs)
            plsc.store_scatter(pys, [rls, io], plsc.load_gather(ybuf, [i0]) - cys)
            plsc.store_scatter(pys, [rls, io + 16], plsc.load_gather(ybuf, [i1]) - cys)
            plsc.store_scatter(pzs, [rls, io], plsc.load_gather(zbuf, [i0]) - czs)
            plsc.store_scatter(pzs, [rls, io + 16], plsc.load_gather(zbuf, [i1]) - czs)
            return 0

        lax.fori_loop(0, _GRP, row, 0)
        return 0

    lax.fori_loop(0, _ROWS_W // _GRP, group, 0)

    pltpu.sync_copy(pxs, px_hbm.at[b, pl.ds(m0, _ROWS_W)])
    pltpu.sync_copy(pys, py_hbm.at[b, pl.ds(m0, _ROWS_W)])
    pltpu.sync_copy(pzs, pz_hbm.at[b, pl.ds(m0, _ROWS_W)])


def _run_knn_sc(D, xb, yb, zb, cxb, cyb, czb):
    f32, i32 = jnp.float32, jnp.int32
    knn = functools.partial(
        pl.kernel,
        mesh=plsc.VectorSubcoreMesh(core_axis_name="c", subcore_axis_name="s"),
        compiler_params=pltpu.CompilerParams(needs_layout_passes=False),
        out_type=[jax.ShapeDtypeStruct((B, M, KNN), f32)] * 3,
        scratch_types=[
            pltpu.VMEM((_GRP, N), f32),
            pltpu.VMEM((N,), f32),
            pltpu.VMEM((N,), f32),
            pltpu.VMEM((N,), f32),
            pltpu.VMEM((_ROWS_W,), f32),
            pltpu.VMEM((_ROWS_W,), f32),
            pltpu.VMEM((_ROWS_W,), f32),
            pltpu.VMEM((_NBKT,), i32),
            pltpu.VMEM((KNN,), i32),
            pltpu.VMEM((16,), f32),
            pltpu.VMEM((16,), i32),
            pltpu.VMEM((_ROWS_W, KNN), f32),
            pltpu.VMEM((_ROWS_W, KNN), f32),
            pltpu.VMEM((_ROWS_W, KNN), f32),
        ],
    )(_knn_sc_body)
    return knn(D, xb, yb, zb, cxb, cyb, czb)


G1 = 32
GR = (B * M) // G1

def _mlp1_body(px_ref, py_ref, pz_ref, w_ref, b_ref, z_ref, s_ref, q_ref):
    px = px_ref[...]
    py = py_ref[...]
    pz = pz_ref[...]
    wx = w_ref[0]
    wy = w_ref[1]
    wz = w_ref[2]
    b = b_ref[...]
    z = (px[:, :, None] * wx[None, None, :]
         + py[:, :, None] * wy[None, None, :]
         + pz[:, :, None] * wz[None, None, :]
         + b[None, :, :])
    z_ref[...] = z

    @pl.when(pl.program_id(0) == 0)
    def _():
        s_ref[...] = jnp.zeros_like(s_ref)
        q_ref[...] = jnp.zeros_like(q_ref)

    s_ref[...] += jnp.sum(z, axis=(0, 1))[None, :]
    q_ref[...] += jnp.sum(z * z, axis=(0, 1))[None, :]


def _run_mlp1(px2, py2, pz2, W1, b1):
    grid = (G1,)
    pspec = pl.BlockSpec((GR, KNN), lambda g: (g, 0))
    wspec = pl.BlockSpec((3, 64), lambda g: (0, 0))
    bspec = pl.BlockSpec((1, 64), lambda g: (0, 0))
    zspec = pl.BlockSpec((GR, KNN, 64), lambda g: (g, 0, 0))
    sspec = pl.BlockSpec((1, 64), lambda g: (0, 0))
    z1, s1, q1 = pl.pallas_call(
        _mlp1_body,
        grid=grid,
        in_specs=[pspec] * 3 + [wspec, bspec],
        out_specs=[zspec, sspec, sspec],
        out_shape=[
            jax.ShapeDtypeStruct((B * M, KNN, 64), jnp.float32),
            jax.ShapeDtypeStruct((1, 64), jnp.float32),
            jax.ShapeDtypeStruct((1, 64), jnp.float32),
        ],
    )(px2, py2, pz2, W1, b1)
    return z1, s1, q1


G2 = 16
R2 = ROWS // G2

def _mlp2_body(z1_ref, s_ref, h_ref, w_ref, b_ref, z2_ref, s2_ref, q2_ref):
    a = jnp.maximum(z1_ref[...] * s_ref[...] + h_ref[...], 0.0)
    z2 = jnp.dot(a, w_ref[...], preferred_element_type=jnp.float32) + b_ref[...]
    z2_ref[...] = z2

    @pl.when(pl.program_id(0) == 0)
    def _():
        s2_ref[...] = jnp.zeros_like(s2_ref)
        q2_ref[...] = jnp.zeros_like(q2_ref)

    s2_ref[...] += jnp.sum(z2, axis=0)[None, :]
    q2_ref[...] += jnp.sum(z2 * z2, axis=0)[None, :]


def _run_mlp2(z1f, sc1, sh1, W2, b2):
    grid = (G2,)
    z2, s2, q2 = pl.pallas_call(
        _mlp2_body,
        grid=grid,
        in_specs=[
            pl.BlockSpec((R2, 64), lambda g: (g, 0)),
            pl.BlockSpec((1, 64), lambda g: (0, 0)),
            pl.BlockSpec((1, 64), lambda g: (0, 0)),
            pl.BlockSpec((64, 128), lambda g: (0, 0)),
            pl.BlockSpec((1, 128), lambda g: (0, 0)),
        ],
        out_specs=[
            pl.BlockSpec((R2, 128), lambda g: (g, 0)),
            pl.BlockSpec((1, 128), lambda g: (0, 0)),
            pl.BlockSpec((1, 128), lambda g: (0, 0)),
        ],
        out_shape=[
            jax.ShapeDtypeStruct((ROWS, 128), jnp.float32),
            jax.ShapeDtypeStruct((1, 128), jnp.float32),
            jax.ShapeDtypeStruct((1, 128), jnp.float32),
        ],
    )(z1f, sc1, sh1, W2, b2)
    return z2, s2, q2


def _mlp3_body(z2_ref, s_ref, h_ref, w_ref, b_ref, a2_ref, s3_ref, q3_ref):
    a2 = jnp.maximum(z2_ref[...] * s_ref[...] + h_ref[...], 0.0)
    a2_ref[...] = a2
    z3 = jnp.dot(a2, w_ref[...], preferred_element_type=jnp.float32) + b_ref[...]

    @pl.when(pl.program_id(0) == 0)
    def _():
        s3_ref[...] = jnp.zeros_like(s3_ref)
        q3_ref[...] = jnp.zeros_like(q3_ref)

    s3_ref[...] += jnp.sum(z3, axis=0)[None, :]
    q3_ref[...] += jnp.sum(z3 * z3, axis=0)[None, :]


def _run_mlp3(z2, sc2, sh2, W3, b3):
    grid = (G2,)
    a2, s3, q3 = pl.pallas_call(
        _mlp3_body,
        grid=grid,
        in_specs=[
            pl.BlockSpec((R2, 128), lambda g: (g, 0)),
            pl.BlockSpec((1, 128), lambda g: (0, 0)),
            pl.BlockSpec((1, 128), lambda g: (0, 0)),
            pl.BlockSpec((128, 384), lambda g: (0, 0)),
            pl.BlockSpec((1, 384), lambda g: (0, 0)),
        ],
        out_specs=[
            pl.BlockSpec((R2, 128), lambda g: (g, 0)),
            pl.BlockSpec((1, 384), lambda g: (0, 0)),
            pl.BlockSpec((1, 384), lambda g: (0, 0)),
        ],
        out_shape=[
            jax.ShapeDtypeStruct((ROWS, 128), jnp.float32),
            jax.ShapeDtypeStruct((1, 384), jnp.float32),
            jax.ShapeDtypeStruct((1, 384), jnp.float32),
        ],
    )(z2, sc2, sh2, W3, b3)
    return a2, s3, q3


def _mlp4_body(a2_ref, s_ref, h_ref, w_ref, b_ref, t_ref):
    a2 = a2_ref[...]
    z3 = jnp.dot(a2, w_ref[...], preferred_element_type=jnp.float32) + b_ref[...]
    y = jnp.maximum(z3 * s_ref[...] + h_ref[...], 0.0)
    y = y.reshape(R2 // KNN, KNN, 384)
    t_ref[...] = jnp.max(y, axis=1)


def _run_mlp4(a2, sc3, sh3, W3, b3):
    grid = (G2,)
    toks = pl.pallas_call(
        _mlp4_body,
        grid=grid,
        in_specs=[
            pl.BlockSpec((R2, 128), lambda g: (g, 0)),
            pl.BlockSpec((1, 384), lambda g: (0, 0)),
            pl.BlockSpec((1, 384), lambda g: (0, 0)),
            pl.BlockSpec((128, 384), lambda g: (0, 0)),
            pl.BlockSpec((1, 384), lambda g: (0, 0)),
        ],
        out_specs=pl.BlockSpec((R2 // KNN, 384), lambda g: (g, 0)),
        out_shape=jax.ShapeDtypeStruct((B * M, 384), jnp.float32),
    )(a2, sc3, sh3, W3, b3)
    return toks


def _bn_coeffs(s, q, g, be):
    mean = s / ROWS
    var = q / ROWS - mean * mean
    sc = g[None, :] / jnp.sqrt(var + 1e-5)
    sh = be[None, :] - mean * sc
    return sc, sh


def kernel(xyz, W1, b1, g1, be1, W2, b2, g2, be2, W3, b3, g3, be3):
    xt = jnp.transpose(xyz, (2, 0, 1))
    cx, cy, cz = _run_fps(xt)
    centers = jnp.stack([cx.T, cy.T, cz.T], axis=-1)
    cxt = jnp.stack([cx.T, cy.T, cz.T])[:, :, :, None]
    D = _run_dist(xt[:, :, None, :], cxt)
    px, py, pz = _run_knn_sc(D, xt[0], xt[1], xt[2],
                             cx.T, cy.T, cz.T)
    px2 = px.reshape(B * M, KNN)
    py2 = py.reshape(B * M, KNN)
    pz2 = pz.reshape(B * M, KNN)
    z1, s1, q1 = _run_mlp1(px2, py2, pz2, W1, b1[None, :])
    sc1, sh1 = _bn_coeffs(s1, q1, g1, be1)
    z1f = z1.reshape(ROWS, 64)
    z2, s2, q2 = _run_mlp2(z1f, sc1, sh1, W2, b2[None, :])
    sc2, sh2 = _bn_coeffs(s2, q2, g2, be2)
    a2, s3, q3 = _run_mlp3(z2, sc2, sh2, W3, b3[None, :])
    sc3, sh3 = _bn_coeffs(s3, q3, g3, be3)
    toks = _run_mlp4(a2, sc3, sh3, W3, b3[None, :])
    tokens = toks.reshape(B, M, 384)
    return (tokens, centers)


# speedup vs baseline: 1.0293x; 1.0293x over previous
import functools

import jax
import jax.numpy as jnp
from jax import lax
from jax.experimental import pallas as pl
from jax.experimental.pallas import tpu as pltpu
from jax.experimental.pallas import tpu_sc as plsc

B = 16
N = 4096
M = 128
KNN = 32
ROWS = B * M * KNN


def _fps_body(x_ref, y_ref, z_ref, cx_ref, cy_ref, cz_ref):
    x = x_ref[...]
    y = y_ref[...]
    z = z_ref[...]
    pt = jax.lax.broadcasted_iota(jnp.int32, (B, N), 1)

    def body(i, carry):
        dist, far = carry
        cm = pt == far[:, None]
        cx = jnp.sum(jnp.where(cm, x, 0.0), axis=1)
        cy = jnp.sum(jnp.where(cm, y, 0.0), axis=1)
        cz = jnp.sum(jnp.where(cm, z, 0.0), axis=1)
        cx_ref[pl.ds(i, 1), :] = cx[None, :]
        cy_ref[pl.ds(i, 1), :] = cy[None, :]
        cz_ref[pl.ds(i, 1), :] = cz[None, :]
        dx = x - cx[:, None]
        dy = y - cy[:, None]
        dz = z - cz[:, None]
        d = dx * dx + dy * dy + dz * dz
        dist = jnp.minimum(dist, d)
        rm = jnp.max(dist, axis=1)
        far = jnp.min(jnp.where(dist == rm[:, None], pt, N), axis=1).astype(jnp.int32)
        return dist, far

    dist0 = jnp.full((B, N), jnp.inf, dtype=jnp.float32)
    far0 = jnp.zeros((B,), dtype=jnp.int32)
    jax.lax.fori_loop(0, M, body, (dist0, far0))


def _run_fps(xt):
    out = pl.pallas_call(
        _fps_body,
        out_shape=[jax.ShapeDtypeStruct((M, B), jnp.float32)] * 3,
    )(xt[0], xt[1], xt[2])
    return out


def _dist_body(x_ref, y_ref, z_ref, cx_ref, cy_ref, cz_ref, d_ref):
    x = x_ref[0]
    y = y_ref[0]
    z = z_ref[0]
    cx = cx_ref[0]
    cy = cy_ref[0]
    cz = cz_ref[0]
    dxx = cx - x
    dyy = cy - y
    dzz = cz - z
    d_ref[0] = dxx * dxx + dyy * dyy + dzz * dzz


def _run_dist(xt, cxt):
    grid = (B,)
    pspec = pl.BlockSpec((1, 1, N), lambda b: (b, 0, 0))
    cspec = pl.BlockSpec((1, M, 1), lambda b: (b, 0, 0))
    return pl.pallas_call(
        _dist_body,
        grid=grid,
        in_specs=[pspec] * 3 + [cspec] * 3,
        out_specs=pl.BlockSpec((1, M, N), lambda b: (b, 0, 0)),
        out_shape=jax.ShapeDtypeStruct((B, M, N), jnp.float32),
    )(xt[0], xt[1], xt[2], cxt[0], cxt[1], cxt[2])


_ROWS_W = 64
_GRP = 8
_NBKT = 4096


def _knn_sc_body(d_hbm, x_hbm, y_hbm, z_hbm, cx_hbm, cy_hbm, cz_hbm,
                 px_hbm, py_hbm, pz_hbm,
                 dbuf, xbuf, ybuf, zbuf, cxb, cyb, czb,
                 hist, idxb, eqv, eqi, pxs, pys, pzs):
    wid = lax.axis_index("s") * 2 + lax.axis_index("c")
    b = wid // 2
    m0 = (wid % 2) * _ROWS_W
    io = lax.iota(jnp.int32, 16)
    ones16 = jnp.ones((16,), jnp.int32)
    zero16 = jnp.zeros((16,), jnp.int32)
    inf16 = jnp.full((16,), jnp.inf, jnp.float32)

    pltpu.sync_copy(x_hbm.at[b], xbuf)
    pltpu.sync_copy(y_hbm.at[b], ybuf)
    pltpu.sync_copy(z_hbm.at[b], zbuf)
    pltpu.sync_copy(cx_hbm.at[b, pl.ds(m0, _ROWS_W)], cxb)
    pltpu.sync_copy(cy_hbm.at[b, pl.ds(m0, _ROWS_W)], cyb)
    pltpu.sync_copy(cz_hbm.at[b, pl.ds(m0, _ROWS_W)], czb)

    def group(g, _g):
        pltpu.sync_copy(d_hbm.at[b, pl.ds(m0 + g * _GRP, _GRP)], dbuf)

        def row(rr, _r):
            rl = g * _GRP + rr

            @plsc.parallel_loop(0, 128, unroll=4, carry=(inf16, inf16))
            def scan1(c, carry):
                a, bb = carry
                va = dbuf[rr, pl.ds(pl.multiple_of(c * 16, 16), 16)]
                vb = dbuf[rr, pl.ds(pl.multiple_of((c + 128) * 16, 16), 16)]
                return jnp.minimum(a, va), jnp.minimum(bb, vb)

            accA, accB = scan1
            minv = jnp.min(jnp.minimum(accA, accB))
            q = jnp.maximum(jnp.max(accA), jnp.max(accB))
            den = jnp.maximum(q - minv, jnp.float32(1e-30))
            s = jnp.full((16,), 4094.0, jnp.float32) / jnp.full(
                (16,), den, jnp.float32)

            @plsc.parallel_loop(0, _NBKT // 16, unroll=8)
            def _clr(c):
                hist[pl.ds(pl.multiple_of(c * 16, 16), 16)] = zero16

            @plsc.parallel_loop(0, 256, unroll=4)
            def _hp(c):
                v = dbuf[rr, pl.ds(pl.multiple_of(c * 16, 16), 16)]
                bi = jnp.minimum((v - minv) * s, 4095.0).astype(jnp.int32)
                plsc.addupdate_scatter(hist, [bi], ones16)

            def condf(carry):
                j, tot, found, beta, below = carry
                return jnp.logical_and(found == 0, j < _NBKT // 16)

            def bodyf(carry):
                j, tot, found, beta, below = carry
                hv = hist[pl.ds(pl.multiple_of(j * 16, 16), 16)]
                pc = plsc.cumsum(hv)
                ta = pc + tot
                cr = ta >= KNN
                anyc = jnp.any(cr)
                l = jnp.min(jnp.where(cr, io, 16))
                below_c = jnp.maximum(jnp.max(jnp.where(cr, 0, ta)), tot)
                beta = jnp.where(anyc, j * 16 + l, beta)
                below = jnp.where(anyc, below_c, below)
                found = jnp.where(anyc, 1, found)
                return j + 1, jnp.max(ta), found, beta, below

            _, _, _, beta, below = lax.while_loop(
                condf, bodyf, (0, 0, 0, 0, 0))

            eqv[...] = inf16
            eqi[...] = zero16

            @plsc.parallel_loop(0, 256, unroll=2,
                                carry=(jnp.int32(0), jnp.int32(0)))
            def _cp(c, carry):
                v = dbuf[rr, pl.ds(pl.multiple_of(c * 16, 16), 16)]
                bi = jnp.minimum((v - minv) * s, 4095.0).astype(jnp.int32)

                def heavy(args):
                    bc, ec = args
                    pidx = io + c * 16
                    mlow = bi < beta
                    pcl = plsc.cumsum(mlow.astype(jnp.int32))
                    plsc.store_scatter(idxb, [pcl - 1 + bc], pidx, mask=mlow)
                    meq = bi == beta
                    pce = plsc.cumsum(meq.astype(jnp.int32))
                    pose = pce - 1 + ec
                    meq2 = jnp.logical_and(meq, pose < 16)
                    plsc.store_scatter(eqv, [pose], v, mask=meq2)
                    plsc.store_scatter(eqi, [pose], pidx, mask=meq2)
                    return bc + jnp.max(pcl), ec + jnp.max(pce)

                return lax.cond(jnp.min(bi) <= beta, heavy,
                                lambda a: a, carry)

            sv, si = plsc.sort_key_val(eqv[...], eqi[...])
            plsc.store_scatter(idxb, [io + below], si,
                               mask=io < (KNN - below))

            rls = jnp.full((16,), rl, jnp.int32)
            cxs = plsc.load_gather(cxb, [rls])
            cys = plsc.load_gather(cyb, [rls])
            czs = plsc.load_gather(czb, [rls])
            i0 = idxb[pl.ds(0, 16)]
            i1 = idxb[pl.ds(16, 16)]
            plsc.store_scatter(pxs, [rls, io], plsc.load_gather(xbuf, [i0]) - cxs)
            plsc.store_scatter(pxs, [rls, io + 16], plsc.load_gather(xbuf, [i1]) - cxs)
            plsc.store_scatter(pys, [rls, io], plsc.load_gather(ybuf, [i0]) - cys)
            plsc.store_scatter(pys, [rls, io + 16], plsc.load_gather(ybuf, [i1]) - cys)
            plsc.store_scatter(pzs, [rls, io], plsc.load_gather(zbuf, [i0]) - czs)
            plsc.store_scatter(pzs, [rls, io + 16], plsc.load_gather(zbuf, [i1]) - czs)
            return 0

        lax.fori_loop(0, _GRP, row, 0)
        return 0

    lax.fori_loop(0, _ROWS_W // _GRP, group, 0)

    pltpu.sync_copy(pxs, px_hbm.at[b, pl.ds(m0, _ROWS_W)])
    pltpu.sync_copy(pys, py_hbm.at[b, pl.ds(m0, _ROWS_W)])
    pltpu.sync_copy(pzs, pz_hbm.at[b, pl.ds(m0, _ROWS_W)])


def _run_knn_sc(D, xb, yb, zb, cxb, cyb, czb):
    f32, i32 = jnp.float32, jnp.int32
    knn = functools.partial(
        pl.kernel,
        mesh=plsc.VectorSubcoreMesh(core_axis_name="c", subcore_axis_name="s"),
        compiler_params=pltpu.CompilerParams(needs_layout_passes=False),
        out_type=[jax.ShapeDtypeStruct((B, M, KNN), f32)] * 3,
        scratch_types=[
            pltpu.VMEM((_GRP, N), f32),
            pltpu.VMEM((N,), f32),
            pltpu.VMEM((N,), f32),
            pltpu.VMEM((N,), f32),
            pltpu.VMEM((_ROWS_W,), f32),
            pltpu.VMEM((_ROWS_W,), f32),
            pltpu.VMEM((_ROWS_W,), f32),
            pltpu.VMEM((_NBKT,), i32),
            pltpu.VMEM((KNN,), i32),
            pltpu.VMEM((16,), f32),
            pltpu.VMEM((16,), i32),
            pltpu.VMEM((_ROWS_W, KNN), f32),
            pltpu.VMEM((_ROWS_W, KNN), f32),
            pltpu.VMEM((_ROWS_W, KNN), f32),
        ],
    )(_knn_sc_body)
    return knn(D, xb, yb, zb, cxb, cyb, czb)


G1 = 32
GR = (B * M) // G1

def _mlp1_body(px_ref, py_ref, pz_ref, w_ref, b_ref, z_ref, s_ref, q_ref):
    px = px_ref[...]
    py = py_ref[...]
    pz = pz_ref[...]
    wx = w_ref[0]
    wy = w_ref[1]
    wz = w_ref[2]
    b = b_ref[...]
    z = (px[:, :, None] * wx[None, None, :]
         + py[:, :, None] * wy[None, None, :]
         + pz[:, :, None] * wz[None, None, :]
         + b[None, :, :])
    z_ref[...] = z

    @pl.when(pl.program_id(0) == 0)
    def _():
        s_ref[...] = jnp.zeros_like(s_ref)
        q_ref[...] = jnp.zeros_like(q_ref)

    s_ref[...] += jnp.sum(z, axis=(0, 1))[None, :]
    q_ref[...] += jnp.sum(z * z, axis=(0, 1))[None, :]


def _run_mlp1(px2, py2, pz2, W1, b1):
    grid = (G1,)
    pspec = pl.BlockSpec((GR, KNN), lambda g: (g, 0))
    wspec = pl.BlockSpec((3, 64), lambda g: (0, 0))
    bspec = pl.BlockSpec((1, 64), lambda g: (0, 0))
    zspec = pl.BlockSpec((GR, KNN, 64), lambda g: (g, 0, 0))
    sspec = pl.BlockSpec((1, 64), lambda g: (0, 0))
    z1, s1, q1 = pl.pallas_call(
        _mlp1_body,
        grid=grid,
        in_specs=[pspec] * 3 + [wspec, bspec],
        out_specs=[zspec, sspec, sspec],
        out_shape=[
            jax.ShapeDtypeStruct((B * M, KNN, 64), jnp.float32),
            jax.ShapeDtypeStruct((1, 64), jnp.float32),
            jax.ShapeDtypeStruct((1, 64), jnp.float32),
        ],
    )(px2, py2, pz2, W1, b1)
    return z1, s1, q1


G2 = 16
R2 = ROWS // G2

def _mlp2_body(z1_ref, s_ref, h_ref, w_ref, b_ref, z2_ref, s2_ref, q2_ref):
    a = jnp.maximum(z1_ref[...] * s_ref[...] + h_ref[...], 0.0)
    z2 = jnp.dot(a, w_ref[...], preferred_element_type=jnp.float32) + b_ref[...]
    z2_ref[...] = z2

    @pl.when(pl.program_id(0) == 0)
    def _():
        s2_ref[...] = jnp.zeros_like(s2_ref)
        q2_ref[...] = jnp.zeros_like(q2_ref)

    s2_ref[...] += jnp.sum(z2, axis=0)[None, :]
    q2_ref[...] += jnp.sum(z2 * z2, axis=0)[None, :]


def _run_mlp2(z1f, sc1, sh1, W2, b2):
    grid = (G2,)
    z2, s2, q2 = pl.pallas_call(
        _mlp2_body,
        grid=grid,
        in_specs=[
            pl.BlockSpec((R2, 64), lambda g: (g, 0)),
            pl.BlockSpec((1, 64), lambda g: (0, 0)),
            pl.BlockSpec((1, 64), lambda g: (0, 0)),
            pl.BlockSpec((64, 128), lambda g: (0, 0)),
            pl.BlockSpec((1, 128), lambda g: (0, 0)),
        ],
        out_specs=[
            pl.BlockSpec((R2, 128), lambda g: (g, 0)),
            pl.BlockSpec((1, 128), lambda g: (0, 0)),
            pl.BlockSpec((1, 128), lambda g: (0, 0)),
        ],
        out_shape=[
            jax.ShapeDtypeStruct((ROWS, 128), jnp.float32),
            jax.ShapeDtypeStruct((1, 128), jnp.float32),
            jax.ShapeDtypeStruct((1, 128), jnp.float32),
        ],
    )(z1f, sc1, sh1, W2, b2)
    return z2, s2, q2


def _mlp3_body(z2_ref, s_ref, h_ref, w_ref, b_ref, a2_ref, s3_ref, q3_ref):
    a2 = jnp.maximum(z2_ref[...] * s_ref[...] + h_ref[...], 0.0)
    a2_ref[...] = a2
    z3 = jnp.dot(a2, w_ref[...], preferred_element_type=jnp.float32) + b_ref[...]

    @pl.when(pl.program_id(0) == 0)
    def _():
        s3_ref[...] = jnp.zeros_like(s3_ref)
        q3_ref[...] = jnp.zeros_like(q3_ref)

    s3_ref[...] += jnp.sum(z3, axis=0)[None, :]
    q3_ref[...] += jnp.sum(z3 * z3, axis=0)[None, :]


def _run_mlp3(z2, sc2, sh2, W3, b3):
    grid = (G2,)
    a2, s3, q3 = pl.pallas_call(
        _mlp3_body,
        grid=grid,
        in_specs=[
            pl.BlockSpec((R2, 128), lambda g: (g, 0)),
            pl.BlockSpec((1, 128), lambda g: (0, 0)),
            pl.BlockSpec((1, 128), lambda g: (0, 0)),
            pl.BlockSpec((128, 384), lambda g: (0, 0)),
            pl.BlockSpec((1, 384), lambda g: (0, 0)),
        ],
        out_specs=[
            pl.BlockSpec((R2, 128), lambda g: (g, 0)),
            pl.BlockSpec((1, 384), lambda g: (0, 0)),
            pl.BlockSpec((1, 384), lambda g: (0, 0)),
        ],
        out_shape=[
            jax.ShapeDtypeStruct((ROWS, 128), jnp.float32),
            jax.ShapeDtypeStruct((1, 384), jnp.float32),
            jax.ShapeDtypeStruct((1, 384), jnp.float32),
        ],
    )(z2, sc2, sh2, W3, b3)
    return a2, s3, q3


def _mlp4_body(a2_ref, s_ref, h_ref, w_ref, b_ref, t_ref):
    a2 = a2_ref[...]
    z3 = jnp.dot(a2, w_ref[...], preferred_element_type=jnp.float32) + b_ref[...]
    y = jnp.maximum(z3 * s_ref[...] + h_ref[...], 0.0)
    y = y.reshape(R2 // KNN, KNN, 384)
    t_ref[...] = jnp.max(y, axis=1)


def _run_mlp4(a2, sc3, sh3, W3, b3):
    grid = (G2,)
    toks = pl.pallas_call(
        _mlp4_body,
        grid=grid,
        in_specs=[
            pl.BlockSpec((R2, 128), lambda g: (g, 0)),
            pl.BlockSpec((1, 384), lambda g: (0, 0)),
            pl.BlockSpec((1, 384), lambda g: (0, 0)),
            pl.BlockSpec((128, 384), lambda g: (0, 0)),
            pl.BlockSpec((1, 384), lambda g: (0, 0)),
        ],
        out_specs=pl.BlockSpec((R2 // KNN, 384), lambda g: (g, 0)),
        out_shape=jax.ShapeDtypeStruct((B * M, 384), jnp.float32),
    )(a2, sc3, sh3, W3, b3)
    return toks


def _bn_coeffs(s, q, g, be):
    mean = s / ROWS
    var = q / ROWS - mean * mean
    sc = g[None, :] / jnp.sqrt(var + 1e-5)
    sh = be[None, :] - mean * sc
    return sc, sh


def kernel(xyz, W1, b1, g1, be1, W2, b2, g2, be2, W3, b3, g3, be3):
    xt = jnp.transpose(xyz, (2, 0, 1))
    cx, cy, cz = _run_fps(xt)
    centers = jnp.stack([cx.T, cy.T, cz.T], axis=-1)
    cxt = jnp.stack([cx.T, cy.T, cz.T])[:, :, :, None]
    D = _run_dist(xt[:, :, None, :], cxt)
    px, py, pz = _run_knn_sc(D, xt[0], xt[1], xt[2],
                             cx.T, cy.T, cz.T)
    px2 = px.reshape(B * M, KNN)
    py2 = py.reshape(B * M, KNN)
    pz2 = pz.reshape(B * M, KNN)
    z1, s1, q1 = _run_mlp1(px2, py2, pz2, W1, b1[None, :])
    sc1, sh1 = _bn_coeffs(s1, q1, g1, be1)
    z1f = z1.reshape(ROWS, 64)
    z2, s2, q2 = _run_mlp2(z1f, sc1, sh1, W2, b2[None, :])
    sc2, sh2 = _bn_coeffs(s2, q2, g2, be2)
    a2, s3, q3 = _run_mlp3(z2, sc2, sh2, W3, b3[None, :])
    sc3, sh3 = _bn_coeffs(s3, q3, g3, be3)
    toks = _run_mlp4(a2, sc3, sh3, W3, b3[None, :])
    tokens = toks.reshape(B, M, 384)
    return (tokens, centers)
